# baseline (speedup 1.0000x reference)
import functools

import jax
import jax.numpy as jnp
from jax import lax
from jax.experimental import pallas as pl
from jax.experimental.pallas import tpu as pltpu
from jax.experimental.pallas import tpu_sc as plsc

N = 512
C = 512
K_SEL = 256
BLK = 8
GRID = N // BLK

SC_LANES = 16
SC_WORKERS = 32


def _soft(logits, u):
    z = logits + (jnp.log(u) - jnp.log1p(-u))
    return jax.nn.sigmoid(z)


def _stream_body(w_ref, b_ref, x_ref, u_ref, s_ref):
    x2 = x_ref[...].reshape(BLK * N, C)
    logits = jnp.dot(x2, w_ref[...]).reshape(BLK, N) + b_ref[0]
    s_ref[...] = _soft(logits, u_ref[...])


_stream = pl.pallas_call(
    _stream_body,
    grid=(GRID,),
    in_specs=[
        pl.BlockSpec((C, 1), lambda g: (0, 0)),
        pl.BlockSpec(memory_space=pltpu.SMEM),
        pl.BlockSpec((BLK, N, C), lambda g: (g, 0, 0)),
        pl.BlockSpec((BLK, N), lambda g: (g, 0)),
    ],
    out_specs=pl.BlockSpec((BLK, N), lambda g: (g, 0)),
    out_shape=jax.ShapeDtypeStruct((N, N), jnp.float32),
)


@functools.lru_cache(maxsize=1)
def _sc_select():
    mesh = plsc.VectorSubcoreMesh(core_axis_name="c", subcore_axis_name="s")

    @functools.partial(
        pl.kernel,
        mesh=mesh,
        out_type=jax.ShapeDtypeStruct((N,), jnp.int32),
        scratch_types=[
            pltpu.VMEM((N, 128), jnp.int32),
            pltpu.VMEM((SC_LANES,), jnp.int32),
        ],
    )
    def sel(s_hbm, thr_hbm, buf_v, thr_v):
        wid = lax.axis_index("s") * 2 + lax.axis_index("c")
        c0 = wid * SC_LANES
        pltpu.sync_copy(s_hbm.at[:, pl.ds((wid // 8) * 128, 128)], buf_v)
        col_off = (wid % 8) * SC_LANES
        zeros = jnp.zeros((SC_LANES,), jnp.int32)

        def bit_step(i, carry):
            prefix, want = carry
            b = 30 - i

            def count_step(r, cnt0):
                k = buf_v[r, pl.ds(col_off, SC_LANES)]
                m = ((k >> (b + 1)) == (prefix >> (b + 1))) \
                    & (((k >> b) & 1) == 0)
                return cnt0 + jnp.where(m, 1, 0)

            cnt0 = lax.fori_loop(0, N, count_step, zeros)
            take1 = want > cnt0
            prefix = jnp.where(take1, prefix | (1 << b), prefix)
            want = jnp.where(take1, want - cnt0, want)
            return prefix, want

        prefix, _ = lax.fori_loop(
            0, 31, bit_step, (zeros, jnp.full((SC_LANES,), K_SEL, jnp.int32)))
        thr_v[...] = prefix
        pltpu.sync_copy(thr_v, thr_hbm.at[pl.ds(c0, SC_LANES)])

    return sel


def _final_body(sa_ref, sb_ref, thra_ref, y_ref):
    sa = sa_ref[...]
    sb = sb_ref[...]
    thra = lax.bitcast_convert_type(thra_ref[...], jnp.float32)
    keys = lax.bitcast_convert_type(sb, jnp.int32)

    def step(i, carry):
        prefix, want = carry
        b = 30 - i
        high_match = (keys >> (b + 1)) == (prefix >> (b + 1))
        bit_is0 = ((keys >> b) & 1) == 0
        cnt0 = jnp.sum(
            jnp.where(high_match & bit_is0, 1, 0).astype(jnp.int32),
            axis=1, keepdims=True)
        take1 = want > cnt0
        prefix = jnp.where(take1, prefix | (1 << b), prefix)
        want = jnp.where(take1, want - cnt0, want)
        return prefix, want

    prefix, _ = lax.fori_loop(
        0, 31, step,
        (jnp.zeros((N, 1), jnp.int32), jnp.full((N, 1), K_SEL, jnp.int32)))
    thrb = lax.bitcast_convert_type(prefix, jnp.float32)
    ha = (sa >= thra).astype(jnp.float32)
    hb = (sb >= thrb).astype(jnp.float32)
    ysum = ha + hb
    y_ref[...] = jnp.where(ysum == 2.0, 1.0, ysum)


_final = pl.pallas_call(
    _final_body,
    in_specs=[
        pl.BlockSpec((N, N), lambda: (0, 0)),
        pl.BlockSpec((N, N), lambda: (0, 0)),
        pl.BlockSpec((1, N), lambda: (0, 0)),
    ],
    out_specs=pl.BlockSpec((N, N), lambda: (0, 0)),
    out_shape=jax.ShapeDtypeStruct((N, N), jnp.float32),
)


def kernel(xab, xba_t, W, b, u_ab, u_ba):
    wcol = W.reshape(C, 1)
    u2a = u_ab.reshape(N, N)
    u2b = u_ba.reshape(N, N)
    sa = _stream(wcol, b, xab, u2a)
    thra_bits = _sc_select()(lax.bitcast_convert_type(sa, jnp.int32))
    sb = _stream(wcol, b, xba_t, u2b)
    y = _final(sa, sb, thra_bits.reshape(1, N))
    return y.reshape(N, N, 1)


# speedup vs baseline: 1.0147x; 1.0147x over previous
import functools

import jax
import jax.numpy as jnp
from jax import lax
from jax.experimental import pallas as pl
from jax.experimental.pallas import tpu as pltpu
from jax.experimental.pallas import tpu_sc as plsc

N = 512
C = 512
K_SEL = 256
BLK = 16
GRID = N // BLK

SC_LANES = 16
SC_WORKERS = 32


def _soft(logits, u):
    z = logits + (jnp.log(u) - jnp.log1p(-u))
    return jax.nn.sigmoid(z)


def _stream_body(w_ref, b_ref, x_ref, u_ref, s_ref):
    x2 = x_ref[...].reshape(BLK * N, C)
    logits = jnp.dot(x2, w_ref[...]).reshape(BLK, N) + b_ref[0]
    s_ref[...] = _soft(logits, u_ref[...])


_stream = pl.pallas_call(
    _stream_body,
    grid=(GRID,),
    in_specs=[
        pl.BlockSpec((C, 1), lambda g: (0, 0)),
        pl.BlockSpec(memory_space=pltpu.SMEM),
        pl.BlockSpec((BLK, N, C), lambda g: (g, 0, 0)),
        pl.BlockSpec((BLK, N), lambda g: (g, 0)),
    ],
    out_specs=pl.BlockSpec((BLK, N), lambda g: (g, 0)),
    out_shape=jax.ShapeDtypeStruct((N, N), jnp.float32),
)


@functools.lru_cache(maxsize=1)
def _sc_select():
    mesh = plsc.VectorSubcoreMesh(core_axis_name="c", subcore_axis_name="s")

    @functools.partial(
        pl.kernel,
        mesh=mesh,
        out_type=jax.ShapeDtypeStruct((N,), jnp.int32),
        scratch_types=[
            pltpu.VMEM((N, 128), jnp.int32),
            pltpu.VMEM((SC_LANES,), jnp.int32),
        ],
    )
    def sel(s_hbm, thr_hbm, buf_v, thr_v):
        wid = lax.axis_index("s") * 2 + lax.axis_index("c")
        c0 = wid * SC_LANES
        pltpu.sync_copy(s_hbm.at[:, pl.ds((wid // 8) * 128, 128)], buf_v)
        col_off = (wid % 8) * SC_LANES
        zeros = jnp.zeros((SC_LANES,), jnp.int32)

        def bit_step(i, carry):
            prefix, want = carry
            b = 30 - i

            def count_step(r, cnt0):
                k = buf_v[r, pl.ds(col_off, SC_LANES)]
                m = ((k >> (b + 1)) == (prefix >> (b + 1))) \
                    & (((k >> b) & 1) == 0)
                return cnt0 + jnp.where(m, 1, 0)

            cnt0 = lax.fori_loop(0, N, count_step, zeros)
            take1 = want > cnt0
            prefix = jnp.where(take1, prefix | (1 << b), prefix)
            want = jnp.where(take1, want - cnt0, want)
            return prefix, want

        prefix, _ = lax.fori_loop(
            0, 31, bit_step, (zeros, jnp.full((SC_LANES,), K_SEL, jnp.int32)))
        thr_v[...] = prefix
        pltpu.sync_copy(thr_v, thr_hbm.at[pl.ds(c0, SC_LANES)])

    return sel


def _final_body(sa_ref, sb_ref, thra_ref, y_ref):
    sa = sa_ref[...]
    sb = sb_ref[...]
    thra = lax.bitcast_convert_type(thra_ref[...], jnp.float32)
    keys = lax.bitcast_convert_type(sb, jnp.int32)

    def step(i, carry):
        prefix, want = carry
        b = 30 - i
        high_match = (keys >> (b + 1)) == (prefix >> (b + 1))
        bit_is0 = ((keys >> b) & 1) == 0
        cnt0 = jnp.sum(
            jnp.where(high_match & bit_is0, 1, 0).astype(jnp.int32),
            axis=1, keepdims=True)
        take1 = want > cnt0
        prefix = jnp.where(take1, prefix | (1 << b), prefix)
        want = jnp.where(take1, want - cnt0, want)
        return prefix, want

    prefix, _ = lax.fori_loop(
        0, 31, step,
        (jnp.zeros((N, 1), jnp.int32), jnp.full((N, 1), K_SEL, jnp.int32)))
    thrb = lax.bitcast_convert_type(prefix, jnp.float32)
    ha = (sa >= thra).astype(jnp.float32)
    hb = (sb >= thrb).astype(jnp.float32)
    ysum = ha + hb
    y_ref[...] = jnp.where(ysum == 2.0, 1.0, ysum)


_final = pl.pallas_call(
    _final_body,
    in_specs=[
        pl.BlockSpec((N, N), lambda: (0, 0)),
        pl.BlockSpec((N, N), lambda: (0, 0)),
        pl.BlockSpec((1, N), lambda: (0, 0)),
    ],
    out_specs=pl.BlockSpec((N, N), lambda: (0, 0)),
    out_shape=jax.ShapeDtypeStruct((N, N), jnp.float32),
)


def kernel(xab, xba_t, W, b, u_ab, u_ba):
    wcol = W.reshape(C, 1)
    u2a = u_ab.reshape(N, N)
    u2b = u_ba.reshape(N, N)
    sa = _stream(wcol, b, xab, u2a)
    thra_bits = _sc_select()(lax.bitcast_convert_type(sa, jnp.int32))
    sb = _stream(wcol, b, xba_t, u2b)
    y = _final(sa, sb, thra_bits.reshape(1, N))
    return y.reshape(N, N, 1)
